# baseline (speedup 1.0000x reference)
import functools

import jax
import jax.numpy as jnp
from jax.experimental import pallas as pl

N_BLK = 1024


def _mmoe_kernel(x_ref, wc_ref, bc_ref, ac_ref, wg_ref, bg_ref, out_ref,
                 *, n_experts, n_tasks, units):
    x = x_ref[...]
    pre = jnp.dot(x, wc_ref[...], preferred_element_type=jnp.float32)
    pre = pre + bc_ref[...]
    eo = jnp.where(pre > 0, pre, ac_ref[...] * pre)
    gl = jnp.dot(x, wg_ref[...], preferred_element_type=jnp.float32)
    gl = gl + bg_ref[...]
    outs = []
    for t in range(n_tasks):
        lt = gl[:, t * n_experts:(t + 1) * n_experts]
        m = jnp.max(lt, axis=1, keepdims=True)
        ex = jnp.exp(lt - m)
        g = ex / jnp.sum(ex, axis=1, keepdims=True)
        acc = g[:, 0:1] * eo[:, 0:units]
        for e in range(1, n_experts):
            acc = acc + g[:, e:e + 1] * eo[:, e * units:(e + 1) * units]
        outs.append(acc)
    out_ref[...] = jnp.concatenate(outs, axis=1)


def kernel(inputs, W_expert, b_expert, alpha, W_gate, b_gate):
    n, d = inputs.shape
    n_experts, _, units = W_expert.shape
    n_tasks = W_gate.shape[0]
    wc = W_expert.transpose(1, 0, 2).reshape(d, n_experts * units)
    bc = b_expert.reshape(1, n_experts * units)
    ac = alpha.reshape(1, n_experts * units)
    wg = W_gate.transpose(1, 0, 2).reshape(d, n_tasks * n_experts)
    bg = b_gate.reshape(1, n_tasks * n_experts)

    grid = (n // N_BLK,)
    out2d = pl.pallas_call(
        functools.partial(_mmoe_kernel, n_experts=n_experts,
                          n_tasks=n_tasks, units=units),
        grid=grid,
        in_specs=[
            pl.BlockSpec((N_BLK, d), lambda i: (i, 0)),
            pl.BlockSpec(wc.shape, lambda i: (0, 0)),
            pl.BlockSpec(bc.shape, lambda i: (0, 0)),
            pl.BlockSpec(ac.shape, lambda i: (0, 0)),
            pl.BlockSpec(wg.shape, lambda i: (0, 0)),
            pl.BlockSpec(bg.shape, lambda i: (0, 0)),
        ],
        out_specs=pl.BlockSpec((N_BLK, n_tasks * units), lambda i: (i, 0)),
        out_shape=jax.ShapeDtypeStruct((n, n_tasks * units), jnp.float32),
    )(inputs, wc, bc, ac, wg, bg)
    return out2d.reshape(n, n_tasks, units)
